# Initial kernel scaffold; baseline (speedup 1.0000x reference)
import jax
import jax.numpy as jnp
from jax.experimental import pallas as pl


def kernel(x, edge_attr, atom_table, bond_table):
    raise NotImplementedError("write your pallas kernel here")


# speedup vs baseline: 1.6362x; 1.6362x over previous
import functools

import jax
import jax.numpy as jnp
from jax import lax
from jax.experimental import pallas as pl
from jax.experimental.pallas import tpu as pltpu
from jax.experimental.pallas import tpu_sc as plsc

NC = 2
NS = 16
NW = NC * NS
CHUNK = 2048
DIM = 16


def _pad_to(n, mult):
    return ((n + mult - 1) // mult) * mult


@functools.lru_cache(maxsize=None)
def _make_gather(n_atom_pad, n_edge_pad, atom_rows, bond_rows):
    a_per_w = n_atom_pad // NW
    e_per_w = n_edge_pad // NW
    a_chunks = a_per_w // CHUNK
    e_chunks = e_per_w // CHUNK

    mesh = plsc.VectorSubcoreMesh(core_axis_name="c", subcore_axis_name="s")

    @functools.partial(
        pl.kernel,
        out_type=(
            jax.ShapeDtypeStruct((n_atom_pad, DIM), jnp.float32),
            jax.ShapeDtypeStruct((n_edge_pad, DIM), jnp.float32),
        ),
        mesh=mesh,
        scratch_types=[
            pltpu.VMEM((CHUNK,), jnp.int32),
            pltpu.VMEM((CHUNK, DIM), jnp.float32),
            pltpu.SemaphoreType.DMA,
        ],
        compiler_params=pltpu.CompilerParams(use_tc_tiling_on_sc=False),
    )
    def gather_kernel(atab, xidx, btab, eidx, xout, eout, idx_v, rows_v, sem):
        wid = lax.axis_index("s") * NC + lax.axis_index("c")

        def run(table, idxs, out, n_chunks, per_w):
            base0 = wid * per_w

            def chunk_body(i, carry):
                base = base0 + i * CHUNK
                pltpu.sync_copy(idxs.at[pl.ds(base, CHUNK)], idx_v)
                pltpu.async_copy(table.at[idx_v], rows_v, sem).wait()
                pltpu.sync_copy(rows_v, out.at[pl.ds(base, CHUNK)])
                return carry

            lax.fori_loop(0, n_chunks, chunk_body, 0)

        run(atab, xidx, xout, a_chunks, a_per_w)
        run(btab, eidx, eout, e_chunks, e_per_w)

    return gather_kernel


def kernel(x, edge_attr, atom_table, bond_table):
    n_atom = x.shape[0] * x.shape[1]
    n_edge = edge_attr.shape[0] * edge_attr.shape[1]
    n_atom_pad = _pad_to(n_atom, NW * CHUNK)
    n_edge_pad = _pad_to(n_edge, NW * CHUNK)

    xf = jnp.pad(x.reshape(-1).astype(jnp.int32), (0, n_atom_pad - n_atom))
    ef = jnp.pad(edge_attr.reshape(-1).astype(jnp.int32), (0, n_edge_pad - n_edge))

    gk = _make_gather(n_atom_pad, n_edge_pad,
                      atom_table.shape[0], bond_table.shape[0])
    xo, eo = gk(atom_table, xf, bond_table, ef)
    x_emb = xo[:n_atom].reshape(x.shape[0], x.shape[1], DIM)
    e_emb = eo[:n_edge].reshape(edge_attr.shape[0], edge_attr.shape[1], DIM)
    return (x_emb, e_emb)
